# baseline (speedup 1.0000x reference)
import functools

import jax
import jax.numpy as jnp
from jax import lax
from jax.experimental import pallas as pl
from jax.experimental.pallas import tpu as pltpu
from jax.experimental.pallas import tpu_sc as plsc

_N = 10000
_E = 320000
_D = 128
_H = 128
_C = 10
_G = 64

_NC, _NS, _L = 2, 16, 16
_NW = _NC * _NS
_NP = 10240
_EPW = _E // _NW
_K = 80
_NCH = _EPW // _K
_RPT = _NP // _NS
_BN = 2048
_GRID = _NP // _BN


_DEG_NB = 5


def _sc_deg_body(dst2_hbm, out_hbm, idx_v, ones_v, zero_v, hist_sh, sem):
    cid = lax.axis_index("c")
    sid = lax.axis_index("s")
    wid = cid * _NS + sid
    z16 = jnp.zeros((_L,), jnp.float32)
    o16 = jnp.ones((_L,), jnp.float32)
    for j in range(_K // _L):
        ones_v[pl.ds(j * _L, _L)] = o16
        zero_v[pl.ds(j * _L, _L)] = z16
    zcps = [pltpu.async_copy(
        zero_v, hist_sh.at[pl.ds(sid * _RPT + t * _K, _K)], sem)
        for t in range(_RPT // _K)]
    pltpu.sync_copy(dst2_hbm.at[wid], idx_v)
    for cp in zcps:
        cp.wait()
    plsc.subcore_barrier()

    def body(gi, carry):
        cps = []
        for b in range(_DEG_NB):
            c = gi * _DEG_NB + b
            cps.append(pltpu.async_copy(
                ones_v, hist_sh.at[idx_v.at[c]], sem, add=True))
        for cp in cps:
            cp.wait()
        return carry

    lax.fori_loop(0, _NCH // _DEG_NB, body, 0)
    plsc.subcore_barrier()
    pltpu.sync_copy(hist_sh.at[pl.ds(sid * _RPT, _RPT)],
                    out_hbm.at[cid, pl.ds(sid * _RPT, _RPT)])


def _build_deg(interpret=False):
    mesh = plsc.VectorSubcoreMesh(core_axis_name="c", subcore_axis_name="s")
    return pl.kernel(
        _sc_deg_body,
        out_type=jax.ShapeDtypeStruct((_NC, _NP), jnp.float32),
        mesh=mesh,
        scratch_types=[
            pltpu.VMEM((_NCH, _K), jnp.int32),
            pltpu.VMEM((_K,), jnp.float32),
            pltpu.VMEM((_K,), jnp.float32),
            pltpu.VMEM_SHARED((_NP,), jnp.float32),
            pltpu.SemaphoreType.DMA,
        ],
        interpret=interpret,
    )


def _sc_scatter_body(g_hbm, src_hbm, dst2_hbm, out_hbm,
                     sidx_v, didx_v, rows_v, acc_sh, gsem):
    cid = lax.axis_index("c")
    sid = lax.axis_index("s")
    wid = cid * _NS + sid
    z16 = jnp.zeros((_L,), jnp.float32)

    def zrow(i, carry):
        for j in range(_H // _L):
            rows_v[0, i, pl.ds(j * _L, _L)] = z16
        return carry

    lax.fori_loop(0, _K, zrow, 0)
    zcps = [pltpu.async_copy(
        rows_v.at[0], acc_sh.at[pl.ds(sid * _RPT + t * _K, _K)], gsem)
        for t in range(_RPT // _K)]
    pltpu.sync_copy(src_hbm.at[pl.ds(wid * _EPW, _EPW)], sidx_v)
    pltpu.sync_copy(dst2_hbm.at[wid], didx_v)
    for cp in zcps:
        cp.wait()
    plsc.subcore_barrier()

    def fire_gather(c, b):
        h = _K // 2
        pltpu.async_copy(g_hbm.at[sidx_v.at[pl.ds(c * _K, h)]],
                         rows_v.at[b, pl.ds(0, h)], gsem)
        pltpu.async_copy(g_hbm.at[sidx_v.at[pl.ds(c * _K + h, h)]],
                         rows_v.at[b, pl.ds(h, h)], gsem)

    def wait_gather(b):
        pltpu.make_async_copy(g_hbm.at[sidx_v.at[pl.ds(0, _K)]], rows_v.at[b], gsem).wait()

    def scatter_sync(c, b):
        pltpu.sync_copy(rows_v.at[b], acc_sh.at[didx_v.at[c]], add=True)

    fire_gather(0, 0)
    fire_gather(1, 1)

    def body(gi, carry):
        for b in range(2):
            c = gi * 2 + b
            wait_gather(b)
            scatter_sync(c, b)

            @pl.when(c + 2 < _NCH)
            def _():
                fire_gather(c + 2, b)
        return carry

    lax.fori_loop(0, _NCH // 2, body, 0)
    wait_gather(0)
    scatter_sync(_NCH - 1, 0)
    plsc.subcore_barrier()
    pltpu.sync_copy(acc_sh.at[pl.ds(sid * _RPT, _RPT)],
                    out_hbm.at[cid, pl.ds(sid * _RPT, _RPT)])


def _build_scatter(interpret=False):
    mesh = plsc.VectorSubcoreMesh(core_axis_name="c", subcore_axis_name="s")
    return pl.kernel(
        _sc_scatter_body,
        out_type=jax.ShapeDtypeStruct((_NC, _NP, _H), jnp.float32),
        mesh=mesh,
        scratch_types=[
            pltpu.VMEM((_EPW,), jnp.int32),
            pltpu.VMEM((_NCH, _K), jnp.int32),
            pltpu.VMEM((2, _K, _H), jnp.float32),
            pltpu.VMEM_SHARED((_NP, _H), jnp.float32),
            pltpu.SemaphoreType.DMA,
        ],
        interpret=interpret,
    )


def _dis_of(deg_blk):
    return lax.rsqrt(deg_blk[:, 0:1] + deg_blk[:, 1:2] + 1.0)


def _tca_body(deg_ref, x_ref, w_ref, g_ref):
    dis = _dis_of(deg_ref[...])
    h = jnp.dot(x_ref[...], w_ref[...], preferred_element_type=jnp.float32)
    g_ref[...] = h * dis


def _build_tca(interpret=False):
    return pl.pallas_call(
        _tca_body,
        grid=(_GRID,),
        in_specs=[
            pl.BlockSpec((_BN, 2), lambda i: (i, 0)),
            pl.BlockSpec((_BN, _D), lambda i: (i, 0)),
            pl.BlockSpec((_D, _H), lambda i: (0, 0)),
        ],
        out_specs=pl.BlockSpec((_BN, _H), lambda i: (i, 0)),
        out_shape=jax.ShapeDtypeStruct((_NP, _H), jnp.float32),
        interpret=interpret,
    )


def _tcb_body(acc_ref, g1_ref, deg_ref, b1_ref, w2_ref, g2_ref):
    dis = _dis_of(deg_ref[...])
    a = acc_ref[...].sum(axis=0)
    g1 = g1_ref[...]
    h1 = jnp.maximum(dis * (a + g1) + b1_ref[...], 0.0)
    g2_ref[...] = jnp.dot(h1, w2_ref[...],
                          preferred_element_type=jnp.float32) * dis


def _build_tcb(interpret=False):
    return pl.pallas_call(
        _tcb_body,
        grid=(_GRID,),
        in_specs=[
            pl.BlockSpec((_NC, _BN, _H), lambda i: (0, i, 0)),
            pl.BlockSpec((_BN, _H), lambda i: (i, 0)),
            pl.BlockSpec((_BN, 2), lambda i: (i, 0)),
            pl.BlockSpec((_H,), lambda i: (0,)),
            pl.BlockSpec((_H, _H), lambda i: (0, 0)),
        ],
        out_specs=pl.BlockSpec((_BN, _H), lambda i: (i, 0)),
        out_shape=jax.ShapeDtypeStruct((_NP, _H), jnp.float32),
        interpret=interpret,
    )


def _tcc_body(acc_ref, g2_ref, deg_ref, b2_ref, batch_ref, wl_ref, bl_ref,
              out_ref, pool_s, cnt_s):
    i = pl.program_id(0)

    @pl.when(i == 0)
    def _():
        pool_s[...] = jnp.zeros((_G, _H), jnp.float32)
        cnt_s[...] = jnp.zeros((_G, _H), jnp.float32)

    dis = _dis_of(deg_ref[...])
    a = acc_ref[...].sum(axis=0)
    h = dis * (a + g2_ref[...]) + b2_ref[...]
    rid = i * _BN + lax.broadcasted_iota(jnp.int32, (_BN, 1), 0)
    h = jnp.where(rid < _N, h, 0.0)
    bt = batch_ref[pl.ds(i, 1), :]
    oh = (lax.broadcasted_iota(jnp.int32, (_G, _BN), 0) == bt)
    oh = oh.astype(jnp.float32)
    pool_s[...] += jnp.dot(oh, h, preferred_element_type=jnp.float32)
    cnt_s[...] += jnp.dot(oh, jnp.ones((_BN, _H), jnp.float32),
                          preferred_element_type=jnp.float32)

    @pl.when(i == _GRID - 1)
    def _():
        mean = pool_s[...] / jnp.maximum(cnt_s[...], 1.0)
        out_ref[...] = jnp.dot(mean, wl_ref[...],
                               preferred_element_type=jnp.float32) + bl_ref[...]


def _build_tcc(interpret=False):
    return pl.pallas_call(
        _tcc_body,
        grid=(_GRID,),
        in_specs=[
            pl.BlockSpec((_NC, _BN, _H), lambda i: (0, i, 0)),
            pl.BlockSpec((_BN, _H), lambda i: (i, 0)),
            pl.BlockSpec((_BN, 2), lambda i: (i, 0)),
            pl.BlockSpec((_H,), lambda i: (0,)),
            pl.BlockSpec((_GRID, _BN), lambda i: (0, 0)),
            pl.BlockSpec((_H, _C), lambda i: (0, 0)),
            pl.BlockSpec((_C,), lambda i: (0,)),
        ],
        out_specs=pl.BlockSpec((_G, _C), lambda i: (0, 0)),
        out_shape=jax.ShapeDtypeStruct((_G, _C), jnp.float32),
        scratch_shapes=[
            pltpu.VMEM((_G, _H), jnp.float32),
            pltpu.VMEM((_G, _H), jnp.float32),
        ],
        interpret=interpret,
    )


def _pipeline(x, edge_index, batch, W1, b1, W2, b2, Wl, bl, interpret=False):
    deg_call = _build_deg(interpret)
    scat_call = _build_scatter(interpret)
    tca = _build_tca(interpret)
    tcb = _build_tcb(interpret)
    tcc = _build_tcc(interpret)

    src1 = edge_index[0]
    dst2 = edge_index[1].reshape(_NW, _NCH, _K)
    bp = jnp.full((_NP,), _G, jnp.int32).at[:_N].set(batch)
    bp = bp.reshape(_GRID, _BN)

    degT = deg_call(dst2).T
    g1 = tca(degT, x, W1)
    acc1 = scat_call(g1, src1, dst2)
    g2 = tcb(acc1, g1, degT, b1, W2)
    acc2 = scat_call(g2, src1, dst2)
    return tcc(acc2, g2, degT, b2, bp, Wl, bl)


def kernel(x, edge_index, batch, W1, b1, W2, b2, Wl, bl):
    return _pipeline(x, edge_index, batch, W1, b1, W2, b2, Wl, bl)


# speedup vs baseline: 1.0012x; 1.0012x over previous
import functools

import jax
import jax.numpy as jnp
from jax import lax
from jax.experimental import pallas as pl
from jax.experimental.pallas import tpu as pltpu
from jax.experimental.pallas import tpu_sc as plsc

_N = 10000
_E = 320000
_D = 128
_H = 128
_C = 10
_G = 64

_NC, _NS, _L = 2, 16, 16
_NW = _NC * _NS
_NP = 10240
_EPW = _E // _NW
_K = 80
_NCH = _EPW // _K
_RPT = _NP // _NS
_BN = 2048
_GRID = _NP // _BN


_DEG_NB = 5


def _sc_deg_body(dst2_hbm, out_hbm, idx_v, ones_v, zero_v, hist_sh, sem):
    cid = lax.axis_index("c")
    sid = lax.axis_index("s")
    wid = cid * _NS + sid
    z16 = jnp.zeros((_L,), jnp.float32)
    o16 = jnp.ones((_L,), jnp.float32)
    for j in range(_K // _L):
        ones_v[pl.ds(j * _L, _L)] = o16
        zero_v[pl.ds(j * _L, _L)] = z16
    zcps = [pltpu.async_copy(
        zero_v, hist_sh.at[pl.ds(sid * _RPT + t * _K, _K)], sem)
        for t in range(_RPT // _K)]
    pltpu.sync_copy(dst2_hbm.at[wid], idx_v)
    for cp in zcps:
        cp.wait()
    plsc.subcore_barrier()

    def body(gi, carry):
        cps = []
        for b in range(_DEG_NB):
            c = gi * _DEG_NB + b
            cps.append(pltpu.async_copy(
                ones_v, hist_sh.at[idx_v.at[c]], sem, add=True))
        for cp in cps:
            cp.wait()
        return carry

    lax.fori_loop(0, _NCH // _DEG_NB, body, 0)
    plsc.subcore_barrier()
    pltpu.sync_copy(hist_sh.at[pl.ds(sid * _RPT, _RPT)],
                    out_hbm.at[cid, pl.ds(sid * _RPT, _RPT)])


def _build_deg(interpret=False):
    mesh = plsc.VectorSubcoreMesh(core_axis_name="c", subcore_axis_name="s")
    return pl.kernel(
        _sc_deg_body,
        out_type=jax.ShapeDtypeStruct((_NC, _NP), jnp.float32),
        mesh=mesh,
        scratch_types=[
            pltpu.VMEM((_NCH, _K), jnp.int32),
            pltpu.VMEM((_K,), jnp.float32),
            pltpu.VMEM((_K,), jnp.float32),
            pltpu.VMEM_SHARED((_NP,), jnp.float32),
            pltpu.SemaphoreType.DMA,
        ],
        interpret=interpret,
    )


def _sc_scatter_body(g_hbm, src_hbm, dst2_hbm, out_hbm,
                     sidx_v, didx_v, rows_v, acc_sh, gsem):
    cid = lax.axis_index("c")
    sid = lax.axis_index("s")
    wid = cid * _NS + sid
    z16 = jnp.zeros((_L,), jnp.float32)

    def zrow(i, carry):
        for j in range(_H // _L):
            rows_v[0, i, pl.ds(j * _L, _L)] = z16
        return carry

    lax.fori_loop(0, _K, zrow, 0)
    zcps = [pltpu.async_copy(
        rows_v.at[0], acc_sh.at[pl.ds(sid * _RPT + t * _K, _K)], gsem)
        for t in range(_RPT // _K)]
    pltpu.sync_copy(src_hbm.at[pl.ds(wid * _EPW, _EPW)], sidx_v)
    pltpu.sync_copy(dst2_hbm.at[wid], didx_v)
    for cp in zcps:
        cp.wait()
    plsc.subcore_barrier()

    def fire_gather(c, b):
        pltpu.async_copy(g_hbm.at[sidx_v.at[pl.ds(c * _K, _K)]], rows_v.at[b], gsem)

    def wait_gather(b):
        pltpu.make_async_copy(g_hbm.at[sidx_v.at[pl.ds(0, _K)]], rows_v.at[b], gsem).wait()

    def scatter_sync(c, b):
        pltpu.sync_copy(rows_v.at[b], acc_sh.at[didx_v.at[c]], add=True)

    fire_gather(0, 0)
    fire_gather(1, 1)

    def body(gi, carry):
        for b in range(2):
            c = gi * 2 + b
            wait_gather(b)
            scatter_sync(c, b)

            @pl.when(c + 2 < _NCH)
            def _():
                fire_gather(c + 2, b)
        return carry

    lax.fori_loop(0, _NCH // 2, body, 0)
    wait_gather(0)
    scatter_sync(_NCH - 1, 0)
    plsc.subcore_barrier()
    pltpu.sync_copy(acc_sh.at[pl.ds(sid * _RPT, _RPT)],
                    out_hbm.at[cid, pl.ds(sid * _RPT, _RPT)])


def _build_scatter(interpret=False):
    mesh = plsc.VectorSubcoreMesh(core_axis_name="c", subcore_axis_name="s")
    return pl.kernel(
        _sc_scatter_body,
        out_type=jax.ShapeDtypeStruct((_NC, _NP, _H), jnp.float32),
        mesh=mesh,
        scratch_types=[
            pltpu.VMEM((_EPW,), jnp.int32),
            pltpu.VMEM((_NCH, _K), jnp.int32),
            pltpu.VMEM((2, _K, _H), jnp.float32),
            pltpu.VMEM_SHARED((_NP, _H), jnp.float32),
            pltpu.SemaphoreType.DMA,
        ],
        interpret=interpret,
    )


def _dis_of(deg_blk):
    return lax.rsqrt(deg_blk[:, 0:1] + deg_blk[:, 1:2] + 1.0)


def _tca_body(deg_ref, x_ref, w_ref, g_ref):
    dis = _dis_of(deg_ref[...])
    h = jnp.dot(x_ref[...], w_ref[...], preferred_element_type=jnp.float32)
    g_ref[...] = h * dis


def _build_tca(interpret=False):
    return pl.pallas_call(
        _tca_body,
        grid=(_GRID,),
        in_specs=[
            pl.BlockSpec((_BN, 2), lambda i: (i, 0)),
            pl.BlockSpec((_BN, _D), lambda i: (i, 0)),
            pl.BlockSpec((_D, _H), lambda i: (0, 0)),
        ],
        out_specs=pl.BlockSpec((_BN, _H), lambda i: (i, 0)),
        out_shape=jax.ShapeDtypeStruct((_NP, _H), jnp.float32),
        interpret=interpret,
    )


def _tcb_body(acc_ref, g1_ref, deg_ref, b1_ref, w2_ref, g2_ref):
    dis = _dis_of(deg_ref[...])
    a = acc_ref[...].sum(axis=0)
    g1 = g1_ref[...]
    h1 = jnp.maximum(dis * (a + g1) + b1_ref[...], 0.0)
    g2_ref[...] = jnp.dot(h1, w2_ref[...],
                          preferred_element_type=jnp.float32) * dis


def _build_tcb(interpret=False):
    return pl.pallas_call(
        _tcb_body,
        grid=(_GRID,),
        in_specs=[
            pl.BlockSpec((_NC, _BN, _H), lambda i: (0, i, 0)),
            pl.BlockSpec((_BN, _H), lambda i: (i, 0)),
            pl.BlockSpec((_BN, 2), lambda i: (i, 0)),
            pl.BlockSpec((_H,), lambda i: (0,)),
            pl.BlockSpec((_H, _H), lambda i: (0, 0)),
        ],
        out_specs=pl.BlockSpec((_BN, _H), lambda i: (i, 0)),
        out_shape=jax.ShapeDtypeStruct((_NP, _H), jnp.float32),
        interpret=interpret,
    )


def _tcc_body(acc_ref, g2_ref, deg_ref, b2_ref, batch_ref, wl_ref, bl_ref,
              out_ref, pool_s, cnt_s):
    i = pl.program_id(0)

    @pl.when(i == 0)
    def _():
        pool_s[...] = jnp.zeros((_G, _H), jnp.float32)
        cnt_s[...] = jnp.zeros((_G, _H), jnp.float32)

    dis = _dis_of(deg_ref[...])
    a = acc_ref[...].sum(axis=0)
    h = dis * (a + g2_ref[...]) + b2_ref[...]
    rid = i * _BN + lax.broadcasted_iota(jnp.int32, (_BN, 1), 0)
    h = jnp.where(rid < _N, h, 0.0)
    bt = batch_ref[pl.ds(i, 1), :]
    oh = (lax.broadcasted_iota(jnp.int32, (_G, _BN), 0) == bt)
    oh = oh.astype(jnp.float32)
    pool_s[...] += jnp.dot(oh, h, preferred_element_type=jnp.float32)
    cnt_s[...] += jnp.dot(oh, jnp.ones((_BN, _H), jnp.float32),
                          preferred_element_type=jnp.float32)

    @pl.when(i == _GRID - 1)
    def _():
        mean = pool_s[...] / jnp.maximum(cnt_s[...], 1.0)
        out_ref[...] = jnp.dot(mean, wl_ref[...],
                               preferred_element_type=jnp.float32) + bl_ref[...]


def _build_tcc(interpret=False):
    return pl.pallas_call(
        _tcc_body,
        grid=(_GRID,),
        in_specs=[
            pl.BlockSpec((_NC, _BN, _H), lambda i: (0, i, 0)),
            pl.BlockSpec((_BN, _H), lambda i: (i, 0)),
            pl.BlockSpec((_BN, 2), lambda i: (i, 0)),
            pl.BlockSpec((_H,), lambda i: (0,)),
            pl.BlockSpec((_GRID, _BN), lambda i: (0, 0)),
            pl.BlockSpec((_H, _C), lambda i: (0, 0)),
            pl.BlockSpec((_C,), lambda i: (0,)),
        ],
        out_specs=pl.BlockSpec((_G, _C), lambda i: (0, 0)),
        out_shape=jax.ShapeDtypeStruct((_G, _C), jnp.float32),
        scratch_shapes=[
            pltpu.VMEM((_G, _H), jnp.float32),
            pltpu.VMEM((_G, _H), jnp.float32),
        ],
        interpret=interpret,
    )


def _pipeline(x, edge_index, batch, W1, b1, W2, b2, Wl, bl, interpret=False):
    deg_call = _build_deg(interpret)
    scat_call = _build_scatter(interpret)
    tca = _build_tca(interpret)
    tcb = _build_tcb(interpret)
    tcc = _build_tcc(interpret)

    src1 = edge_index[0]
    dst2 = edge_index[1].reshape(_NW, _NCH, _K)
    bp = jnp.full((_NP,), _G, jnp.int32).at[:_N].set(batch)
    bp = bp.reshape(_GRID, _BN)

    degT = deg_call(dst2).T
    g1 = tca(degT, x, W1)
    acc1 = scat_call(g1, src1, dst2)
    g2 = tcb(acc1, g1, degT, b1, W2)
    acc2 = scat_call(g2, src1, dst2)
    return tcc(acc2, g2, degT, b2, bp, Wl, bl)


def kernel(x, edge_index, batch, W1, b1, W2, b2, Wl, bl):
    return _pipeline(x, edge_index, batch, W1, b1, W2, b2, Wl, bl)
